# baseline (speedup 1.0000x reference)
import functools

import jax
import jax.numpy as jnp
from jax import lax
from jax.experimental import pallas as pl
from jax.experimental.pallas import tpu as pltpu
from jax.experimental.pallas import tpu_sc as plsc

N = 10000
E = 320000
D = 128
H = 16
C = 16
G = 64

NC = 2
NS = 16
NW = NC * NS
CHUNK = 128
GCH = 8
GSIZE = GCH * CHUNK
NBUF = 4
AHEAD = NBUF - 2
NCHUNK = 80
NSUP = NCHUNK // GCH
E_PAD = NW * NCHUNK * CHUNK
N_PAD = 10112
ROWS_PER_TILE = N_PAD // NS
ROW_BLK = 2000
N_BLKS = N // ROW_BLK


def _k1_body(x_ref, w_ref, y_ref):
    y_ref[...] = jnp.dot(x_ref[...], w_ref[...],
                         preferred_element_type=jnp.float32)


def _project(x, W1):
    return pl.pallas_call(
        _k1_body,
        grid=(N_BLKS,),
        in_specs=[
            pl.BlockSpec((ROW_BLK, D), lambda i: (i, 0)),
            pl.BlockSpec((D, H), lambda i: (0, 0)),
        ],
        out_specs=pl.BlockSpec((ROW_BLK, H), lambda i: (i, 0)),
        out_shape=jax.ShapeDtypeStruct((N, H), jnp.float32),
    )(x, W1)


def _sc_edge_segment_sum(table, src3d, dst3d, zeros_init):
    mesh = plsc.VectorSubcoreMesh(core_axis_name="c", subcore_axis_name="s")

    @functools.partial(
        pl.kernel,
        out_type=jax.ShapeDtypeStruct((NC, N_PAD, H), jnp.float32),
        mesh=mesh,
        compiler_params=pltpu.CompilerParams(use_tc_tiling_on_sc=False),
        scratch_types=[
            pltpu.VMEM((NSUP, GSIZE), jnp.int32),
            pltpu.VMEM((NSUP, GSIZE), jnp.int32),
            pltpu.VMEM((NBUF, GSIZE, H), jnp.float32),
            pltpu.VMEM_SHARED((N_PAD, H), jnp.float32),
            pltpu.SemaphoreType.DMA,
            pltpu.SemaphoreType.DMA,
        ],
    )
    def k(table_hbm, src_hbm, dst_hbm, zeros_hbm, out_hbm,
          src_v, dst_v, rows_v, acc_sh, gsem, ssem):
        c = lax.axis_index("c")
        s = lax.axis_index("s")
        w = c * NS + s

        pltpu.sync_copy(src_hbm.at[w], src_v)
        pltpu.sync_copy(dst_hbm.at[w], dst_v)

        @pl.when(s == 0)
        def _():
            pltpu.sync_copy(zeros_hbm, acc_sh)

        plsc.subcore_barrier()

        def gather_start(j):
            pltpu.async_copy(table_hbm.at[src_v.at[j]],
                             rows_v.at[j % NBUF], gsem)

        def scatter_drain():
            pltpu.make_async_copy(rows_v.at[0],
                                  acc_sh.at[dst_v.at[0]],
                                  ssem).wait()

        for j in range(AHEAD):
            gather_start(j)

        def body(j, carry):
            b = j % NBUF
            pltpu.make_async_copy(table_hbm.at[src_v.at[j]],
                                  rows_v.at[b], gsem).wait()
            pltpu.async_copy(rows_v.at[b],
                             acc_sh.at[dst_v.at[j]],
                             ssem, add=True)
            @pl.when(j >= 2)
            def _():
                scatter_drain()

            @pl.when(j + AHEAD < NSUP)
            def _():
                gather_start(j + AHEAD)
            return carry

        lax.fori_loop(0, NSUP, body, 0)
        scatter_drain()
        scatter_drain()

        plsc.subcore_barrier()

        base = s * ROWS_PER_TILE
        pltpu.sync_copy(acc_sh.at[pl.ds(base, ROWS_PER_TILE)],
                        out_hbm.at[c].at[pl.ds(base, ROWS_PER_TILE)])

    return k(table, src3d, dst3d, zeros_init)


def _onehot(bt):
    gids = lax.broadcasted_iota(jnp.int32, (G, ROW_BLK), 0)
    return (bt[None, :] == gids).astype(jnp.float32)


def _k3_body(y_ref, a0_ref, a1_ref, b1_ref, w2_ref, batch_ref, z_ref, p1_ref):
    x1 = y_ref[...] + a0_ref[...] + a1_ref[...] + b1_ref[...]
    h = jnp.maximum(x1, 0.0)
    z_ref[...] = jnp.dot(h, w2_ref[...], preferred_element_type=jnp.float32)
    oh = _onehot(batch_ref[0, 0, :])
    p1_blk = jnp.dot(oh, x1, preferred_element_type=jnp.float32)
    i = pl.program_id(0)

    @pl.when(i == 0)
    def _():
        p1_ref[...] = p1_blk

    @pl.when(i > 0)
    def _():
        p1_ref[...] += p1_blk


def _fuse1(y, a0, a1, b1, W2, batch3d):
    return pl.pallas_call(
        _k3_body,
        grid=(N_BLKS,),
        in_specs=[
            pl.BlockSpec((ROW_BLK, H), lambda i: (i, 0)),
            pl.BlockSpec((ROW_BLK, H), lambda i: (i, 0)),
            pl.BlockSpec((ROW_BLK, H), lambda i: (i, 0)),
            pl.BlockSpec((1, H), lambda i: (0, 0)),
            pl.BlockSpec((H, C), lambda i: (0, 0)),
            pl.BlockSpec((1, 1, ROW_BLK), lambda i: (i, 0, 0)),
        ],
        out_specs=[
            pl.BlockSpec((ROW_BLK, C), lambda i: (i, 0)),
            pl.BlockSpec((G, H), lambda i: (0, 0)),
        ],
        out_shape=[
            jax.ShapeDtypeStruct((N, C), jnp.float32),
            jax.ShapeDtypeStruct((G, H), jnp.float32),
        ],
    )(y, a0, a1, b1, W2, batch3d)


def _k5_body(z_ref, a0_ref, a1_ref, b2_ref, batch_ref, p1_ref, o_ref, acc_ref):
    x2 = z_ref[...] + a0_ref[...] + a1_ref[...] + b2_ref[...]
    oh = _onehot(batch_ref[0, 0, :])
    p2_blk = jnp.dot(oh, x2, preferred_element_type=jnp.float32)
    i = pl.program_id(0)

    @pl.when(i == 0)
    def _():
        acc_ref[...] = p2_blk

    @pl.when(i > 0)
    def _():
        acc_ref[...] += p2_blk

    @pl.when(i == pl.num_programs(0) - 1)
    def _():
        pcat = jnp.concatenate([p1_ref[...], acc_ref[...]], axis=1)
        m = jnp.max(pcat, axis=1, keepdims=True)
        lse = jnp.log(jnp.sum(jnp.exp(pcat - m), axis=1, keepdims=True)) + m
        o_ref[...] = pcat - lse


def _fuse2(z, a0, a1, b2, batch3d, pool1):
    return pl.pallas_call(
        _k5_body,
        grid=(N_BLKS,),
        in_specs=[
            pl.BlockSpec((ROW_BLK, C), lambda i: (i, 0)),
            pl.BlockSpec((ROW_BLK, C), lambda i: (i, 0)),
            pl.BlockSpec((ROW_BLK, C), lambda i: (i, 0)),
            pl.BlockSpec((1, C), lambda i: (0, 0)),
            pl.BlockSpec((1, 1, ROW_BLK), lambda i: (i, 0, 0)),
            pl.BlockSpec((G, H), lambda i: (0, 0)),
        ],
        out_specs=pl.BlockSpec((G, H + C), lambda i: (0, 0)),
        out_shape=jax.ShapeDtypeStruct((G, H + C), jnp.float32),
        scratch_shapes=[pltpu.VMEM((G, C), jnp.float32)],
    )(z, a0, a1, b2, batch3d, pool1)


def kernel(x, edge_index, batch, W1, b1, W2, b2):
    src = edge_index[0]
    dst = edge_index[1]

    pad = E_PAD - E
    src_p = jnp.concatenate([src, jnp.zeros((pad,), jnp.int32)])
    dst_p = jnp.concatenate([dst, jnp.full((pad,), N, jnp.int32)])
    src3d = src_p.reshape(NW, NSUP, GSIZE)
    dst3d = dst_p.reshape(NW, NSUP, GSIZE)
    zeros_init = jnp.zeros((N_PAD, H), jnp.float32)
    batch3d = batch.reshape(N_BLKS, 1, ROW_BLK)
    b1r = b1.reshape(1, H)
    b2r = b2.reshape(1, C)

    y = _project(x, W1)

    aggy = _sc_edge_segment_sum(y, src3d, dst3d, zeros_init)
    z, pool1 = _fuse1(y, aggy[0, :N], aggy[1, :N], b1r, W2, batch3d)

    aggz = _sc_edge_segment_sum(z, src3d, dst3d, zeros_init)
    out = _fuse2(z, aggz[0, :N], aggz[1, :N], b2r, batch3d, pool1)
    return out


# speedup vs baseline: 1.0125x; 1.0125x over previous
import functools

import jax
import jax.numpy as jnp
from jax import lax
from jax.experimental import pallas as pl
from jax.experimental.pallas import tpu as pltpu
from jax.experimental.pallas import tpu_sc as plsc

N = 10000
E = 320000
D = 128
H = 16
C = 16
G = 64

NC = 2
NS = 16
NW = NC * NS
CHUNK = 128
GCH = 2
GSIZE = GCH * CHUNK
NBUF = 4
AHEAD = NBUF - 2
NCHUNK = 80
NSUP = NCHUNK // GCH
E_PAD = NW * NCHUNK * CHUNK
N_PAD = 10112
ROWS_PER_TILE = N_PAD // NS
ROW_BLK = 2000
N_BLKS = N // ROW_BLK


def _k1_body(x_ref, w_ref, y_ref):
    y_ref[...] = jnp.dot(x_ref[...], w_ref[...],
                         preferred_element_type=jnp.float32)


def _project(x, W1):
    return pl.pallas_call(
        _k1_body,
        grid=(N_BLKS,),
        in_specs=[
            pl.BlockSpec((ROW_BLK, D), lambda i: (i, 0)),
            pl.BlockSpec((D, H), lambda i: (0, 0)),
        ],
        out_specs=pl.BlockSpec((ROW_BLK, H), lambda i: (i, 0)),
        out_shape=jax.ShapeDtypeStruct((N, H), jnp.float32),
    )(x, W1)


def _sc_edge_segment_sum(table, src3d, dst3d, zeros_init):
    mesh = plsc.VectorSubcoreMesh(core_axis_name="c", subcore_axis_name="s")

    @functools.partial(
        pl.kernel,
        out_type=jax.ShapeDtypeStruct((NC, N_PAD, H), jnp.float32),
        mesh=mesh,
        compiler_params=pltpu.CompilerParams(use_tc_tiling_on_sc=False),
        scratch_types=[
            pltpu.VMEM((NSUP, GSIZE), jnp.int32),
            pltpu.VMEM((NSUP, GSIZE), jnp.int32),
            pltpu.VMEM((NBUF, GSIZE, H), jnp.float32),
            pltpu.VMEM_SHARED((N_PAD, H), jnp.float32),
            pltpu.SemaphoreType.DMA,
            pltpu.SemaphoreType.DMA,
        ],
    )
    def k(table_hbm, src_hbm, dst_hbm, zeros_hbm, out_hbm,
          src_v, dst_v, rows_v, acc_sh, gsem, ssem):
        c = lax.axis_index("c")
        s = lax.axis_index("s")
        w = c * NS + s

        pltpu.sync_copy(src_hbm.at[w], src_v)
        pltpu.sync_copy(dst_hbm.at[w], dst_v)

        @pl.when(s == 0)
        def _():
            pltpu.sync_copy(zeros_hbm, acc_sh)

        plsc.subcore_barrier()

        def gather_start(j):
            pltpu.async_copy(table_hbm.at[src_v.at[j]],
                             rows_v.at[j % NBUF], gsem)

        def scatter_drain():
            pltpu.make_async_copy(rows_v.at[0],
                                  acc_sh.at[dst_v.at[0]],
                                  ssem).wait()

        for j in range(AHEAD):
            gather_start(j)

        def body(j, carry):
            b = j % NBUF
            pltpu.make_async_copy(table_hbm.at[src_v.at[j]],
                                  rows_v.at[b], gsem).wait()
            pltpu.async_copy(rows_v.at[b],
                             acc_sh.at[dst_v.at[j]],
                             ssem, add=True)
            @pl.when(j >= 2)
            def _():
                scatter_drain()

            @pl.when(j + AHEAD < NSUP)
            def _():
                gather_start(j + AHEAD)
            return carry

        lax.fori_loop(0, NSUP, body, 0)
        scatter_drain()
        scatter_drain()

        plsc.subcore_barrier()

        base = s * ROWS_PER_TILE
        pltpu.sync_copy(acc_sh.at[pl.ds(base, ROWS_PER_TILE)],
                        out_hbm.at[c].at[pl.ds(base, ROWS_PER_TILE)])

    return k(table, src3d, dst3d, zeros_init)


def _onehot(bt):
    gids = lax.broadcasted_iota(jnp.int32, (G, ROW_BLK), 0)
    return (bt[None, :] == gids).astype(jnp.float32)


def _k3_body(y_ref, a0_ref, a1_ref, b1_ref, w2_ref, batch_ref, z_ref, p1_ref):
    x1 = y_ref[...] + a0_ref[...] + a1_ref[...] + b1_ref[...]
    h = jnp.maximum(x1, 0.0)
    z_ref[...] = jnp.dot(h, w2_ref[...], preferred_element_type=jnp.float32)
    oh = _onehot(batch_ref[0, 0, :])
    p1_blk = jnp.dot(oh, x1, preferred_element_type=jnp.float32)
    i = pl.program_id(0)

    @pl.when(i == 0)
    def _():
        p1_ref[...] = p1_blk

    @pl.when(i > 0)
    def _():
        p1_ref[...] += p1_blk


def _fuse1(y, a0, a1, b1, W2, batch3d):
    return pl.pallas_call(
        _k3_body,
        grid=(N_BLKS,),
        in_specs=[
            pl.BlockSpec((ROW_BLK, H), lambda i: (i, 0)),
            pl.BlockSpec((ROW_BLK, H), lambda i: (i, 0)),
            pl.BlockSpec((ROW_BLK, H), lambda i: (i, 0)),
            pl.BlockSpec((1, H), lambda i: (0, 0)),
            pl.BlockSpec((H, C), lambda i: (0, 0)),
            pl.BlockSpec((1, 1, ROW_BLK), lambda i: (i, 0, 0)),
        ],
        out_specs=[
            pl.BlockSpec((ROW_BLK, C), lambda i: (i, 0)),
            pl.BlockSpec((G, H), lambda i: (0, 0)),
        ],
        out_shape=[
            jax.ShapeDtypeStruct((N, C), jnp.float32),
            jax.ShapeDtypeStruct((G, H), jnp.float32),
        ],
    )(y, a0, a1, b1, W2, batch3d)


def _k5_body(z_ref, a0_ref, a1_ref, b2_ref, batch_ref, p1_ref, o_ref, acc_ref):
    x2 = z_ref[...] + a0_ref[...] + a1_ref[...] + b2_ref[...]
    oh = _onehot(batch_ref[0, 0, :])
    p2_blk = jnp.dot(oh, x2, preferred_element_type=jnp.float32)
    i = pl.program_id(0)

    @pl.when(i == 0)
    def _():
        acc_ref[...] = p2_blk

    @pl.when(i > 0)
    def _():
        acc_ref[...] += p2_blk

    @pl.when(i == pl.num_programs(0) - 1)
    def _():
        pcat = jnp.concatenate([p1_ref[...], acc_ref[...]], axis=1)
        m = jnp.max(pcat, axis=1, keepdims=True)
        lse = jnp.log(jnp.sum(jnp.exp(pcat - m), axis=1, keepdims=True)) + m
        o_ref[...] = pcat - lse


def _fuse2(z, a0, a1, b2, batch3d, pool1):
    return pl.pallas_call(
        _k5_body,
        grid=(N_BLKS,),
        in_specs=[
            pl.BlockSpec((ROW_BLK, C), lambda i: (i, 0)),
            pl.BlockSpec((ROW_BLK, C), lambda i: (i, 0)),
            pl.BlockSpec((ROW_BLK, C), lambda i: (i, 0)),
            pl.BlockSpec((1, C), lambda i: (0, 0)),
            pl.BlockSpec((1, 1, ROW_BLK), lambda i: (i, 0, 0)),
            pl.BlockSpec((G, H), lambda i: (0, 0)),
        ],
        out_specs=pl.BlockSpec((G, H + C), lambda i: (0, 0)),
        out_shape=jax.ShapeDtypeStruct((G, H + C), jnp.float32),
        scratch_shapes=[pltpu.VMEM((G, C), jnp.float32)],
    )(z, a0, a1, b2, batch3d, pool1)


def kernel(x, edge_index, batch, W1, b1, W2, b2):
    src = edge_index[0]
    dst = edge_index[1]

    pad = E_PAD - E
    src_p = jnp.concatenate([src, jnp.zeros((pad,), jnp.int32)])
    dst_p = jnp.concatenate([dst, jnp.full((pad,), N, jnp.int32)])
    src3d = src_p.reshape(NW, NSUP, GSIZE)
    dst3d = dst_p.reshape(NW, NSUP, GSIZE)
    zeros_init = jnp.zeros((N_PAD, H), jnp.float32)
    batch3d = batch.reshape(N_BLKS, 1, ROW_BLK)
    b1r = b1.reshape(1, H)
    b2r = b2.reshape(1, C)

    y = _project(x, W1)

    aggy = _sc_edge_segment_sum(y, src3d, dst3d, zeros_init)
    z, pool1 = _fuse1(y, aggy[0, :N], aggy[1, :N], b1r, W2, batch3d)

    aggz = _sc_edge_segment_sum(z, src3d, dst3d, zeros_init)
    out = _fuse2(z, aggz[0, :N], aggz[1, :N], b2r, batch3d, pool1)
    return out


# speedup vs baseline: 1.6702x; 1.6496x over previous
import functools

import jax
import jax.numpy as jnp
from jax import lax
from jax.experimental import pallas as pl
from jax.experimental.pallas import tpu as pltpu
from jax.experimental.pallas import tpu_sc as plsc

N = 10000
E = 320000
D = 128
H = 16
C = 16
G = 64

NC = 2
NS = 16
NW = NC * NS
CHUNK = 128
ECHUNKS = E // CHUNK
FULL = -(-ECHUNKS // NW)
LAST = ECHUNKS - FULL * (NW - 1)
NBUF = 6
AHEAD = NBUF - 2
N_PAD = 10112
ROWS_PER_TILE = N_PAD // NS
ROW_BLK = 5000
N_BLKS = N // ROW_BLK


def _k1_body(x_ref, w_ref, y_ref):
    y_ref[...] = jnp.dot(x_ref[...], w_ref[...],
                         preferred_element_type=jnp.float32)


def _project(x, W1):
    return pl.pallas_call(
        _k1_body,
        grid=(N_BLKS,),
        in_specs=[
            pl.BlockSpec((ROW_BLK, D), lambda i: (i, 0)),
            pl.BlockSpec((D, H), lambda i: (0, 0)),
        ],
        out_specs=pl.BlockSpec((ROW_BLK, H), lambda i: (i, 0)),
        out_shape=jax.ShapeDtypeStruct((N, H), jnp.float32),
    )(x, W1)


def _sc_edge_segment_sum(table, edges3d, zeros_init):
    mesh = plsc.VectorSubcoreMesh(core_axis_name="c", subcore_axis_name="s")

    @functools.partial(
        pl.kernel,
        out_type=jax.ShapeDtypeStruct((NC, N_PAD, H), jnp.float32),
        mesh=mesh,
        compiler_params=pltpu.CompilerParams(use_tc_tiling_on_sc=False),
        scratch_types=[
            pltpu.VMEM((FULL, CHUNK), jnp.int32),
            pltpu.VMEM((FULL, CHUNK), jnp.int32),
            pltpu.VMEM((NBUF, CHUNK, H), jnp.float32),
            pltpu.VMEM_SHARED((N_PAD, H), jnp.float32),
            pltpu.SemaphoreType.DMA,
            pltpu.SemaphoreType.DMA,
        ],
    )
    def k(table_hbm, edge_hbm, zeros_hbm, out_hbm,
          src_v, dst_v, rows_v, acc_sh, gsem, ssem):
        c = lax.axis_index("c")
        s = lax.axis_index("s")
        w = c * NS + s
        nchunks = jnp.where(w == NW - 1, LAST, FULL)
        cbase = w * FULL

        @pl.when(w < NW - 1)
        def _():
            pltpu.sync_copy(edge_hbm.at[0].at[pl.ds(cbase, FULL)], src_v)
            pltpu.sync_copy(edge_hbm.at[1].at[pl.ds(cbase, FULL)], dst_v)

        @pl.when(w == NW - 1)
        def _():
            pltpu.sync_copy(edge_hbm.at[0].at[pl.ds(cbase, LAST)],
                            src_v.at[pl.ds(0, LAST)])
            pltpu.sync_copy(edge_hbm.at[1].at[pl.ds(cbase, LAST)],
                            dst_v.at[pl.ds(0, LAST)])

        @pl.when(s == 0)
        def _():
            pltpu.sync_copy(zeros_hbm, acc_sh)

        plsc.subcore_barrier()

        def gather_start(j):
            pltpu.async_copy(table_hbm.at[src_v.at[j]],
                             rows_v.at[j % NBUF], gsem)

        def scatter_drain():
            pltpu.make_async_copy(rows_v.at[0], acc_sh.at[dst_v.at[0]],
                                  ssem).wait()

        for j in range(AHEAD):
            gather_start(j)

        def body(j, carry):
            b = j % NBUF
            pltpu.make_async_copy(table_hbm.at[src_v.at[j]],
                                  rows_v.at[b], gsem).wait()
            pltpu.async_copy(rows_v.at[b], acc_sh.at[dst_v.at[j]],
                             ssem, add=True)
            @pl.when(j >= 2)
            def _():
                scatter_drain()

            @pl.when(j + AHEAD < nchunks)
            def _():
                gather_start(j + AHEAD)
            return carry

        lax.fori_loop(0, nchunks, body, 0)
        scatter_drain()
        scatter_drain()

        plsc.subcore_barrier()

        base = s * ROWS_PER_TILE
        pltpu.sync_copy(acc_sh.at[pl.ds(base, ROWS_PER_TILE)],
                        out_hbm.at[c].at[pl.ds(base, ROWS_PER_TILE)])

    return k(table, edges3d, zeros_init)


def _onehot(bt):
    gids = lax.broadcasted_iota(jnp.int32, (G, ROW_BLK), 0)
    return (bt[None, :] == gids).astype(jnp.float32)


def _k3_body(y_ref, a_ref, b1_ref, w2_ref, batch_ref, z_ref, p1_ref):
    x1 = y_ref[...] + a_ref[0] + a_ref[1] + b1_ref[...]
    h = jnp.maximum(x1, 0.0)
    z_ref[...] = jnp.dot(h, w2_ref[...], preferred_element_type=jnp.float32)
    oh = _onehot(batch_ref[0, 0, :])
    p1_blk = jnp.dot(oh, x1, preferred_element_type=jnp.float32)
    i = pl.program_id(0)

    @pl.when(i == 0)
    def _():
        p1_ref[...] = p1_blk

    @pl.when(i > 0)
    def _():
        p1_ref[...] += p1_blk


def _fuse1(y, agg, b1, W2, batch3d):
    return pl.pallas_call(
        _k3_body,
        grid=(N_BLKS,),
        in_specs=[
            pl.BlockSpec((ROW_BLK, H), lambda i: (i, 0)),
            pl.BlockSpec((NC, ROW_BLK, H), lambda i: (0, i, 0)),
            pl.BlockSpec((1, H), lambda i: (0, 0)),
            pl.BlockSpec((H, C), lambda i: (0, 0)),
            pl.BlockSpec((1, 1, ROW_BLK), lambda i: (i, 0, 0)),
        ],
        out_specs=[
            pl.BlockSpec((ROW_BLK, C), lambda i: (i, 0)),
            pl.BlockSpec((G, H), lambda i: (0, 0)),
        ],
        out_shape=[
            jax.ShapeDtypeStruct((N, C), jnp.float32),
            jax.ShapeDtypeStruct((G, H), jnp.float32),
        ],
    )(y, agg, b1, W2, batch3d)


def _k5_body(z_ref, a_ref, b2_ref, batch_ref, p1_ref, o_ref, acc_ref):
    x2 = z_ref[...] + a_ref[0] + a_ref[1] + b2_ref[...]
    oh = _onehot(batch_ref[0, 0, :])
    p2_blk = jnp.dot(oh, x2, preferred_element_type=jnp.float32)
    i = pl.program_id(0)

    @pl.when(i == 0)
    def _():
        acc_ref[...] = p2_blk

    @pl.when(i > 0)
    def _():
        acc_ref[...] += p2_blk

    @pl.when(i == pl.num_programs(0) - 1)
    def _():
        pcat = jnp.concatenate([p1_ref[...], acc_ref[...]], axis=1)
        m = jnp.max(pcat, axis=1, keepdims=True)
        lse = jnp.log(jnp.sum(jnp.exp(pcat - m), axis=1, keepdims=True)) + m
        o_ref[...] = pcat - lse


def _fuse2(z, agg, b2, batch3d, pool1):
    return pl.pallas_call(
        _k5_body,
        grid=(N_BLKS,),
        in_specs=[
            pl.BlockSpec((ROW_BLK, C), lambda i: (i, 0)),
            pl.BlockSpec((NC, ROW_BLK, C), lambda i: (0, i, 0)),
            pl.BlockSpec((1, C), lambda i: (0, 0)),
            pl.BlockSpec((1, 1, ROW_BLK), lambda i: (i, 0, 0)),
            pl.BlockSpec((G, H), lambda i: (0, 0)),
        ],
        out_specs=pl.BlockSpec((G, H + C), lambda i: (0, 0)),
        out_shape=jax.ShapeDtypeStruct((G, H + C), jnp.float32),
        scratch_shapes=[pltpu.VMEM((G, C), jnp.float32)],
    )(z, agg, b2, batch3d, pool1)


def kernel(x, edge_index, batch, W1, b1, W2, b2):
    edges3d = edge_index.reshape(2, ECHUNKS, CHUNK)
    zeros_init = jnp.zeros((N_PAD, H), jnp.float32)
    batch3d = batch.reshape(N_BLKS, 1, ROW_BLK)
    b1r = b1.reshape(1, H)
    b2r = b2.reshape(1, C)

    y = _project(x, W1)

    aggy = _sc_edge_segment_sum(y, edges3d, zeros_init)
    z, pool1 = _fuse1(y, aggy, b1r, W2, batch3d)

    aggz = _sc_edge_segment_sum(z, edges3d, zeros_init)
    out = _fuse2(z, aggz, b2r, batch3d, pool1)
    return out
